# Initial kernel scaffold; baseline (speedup 1.0000x reference)
import jax
import jax.numpy as jnp
from jax.experimental import pallas as pl


def kernel(indices, table):
    raise NotImplementedError("write your pallas kernel here")


# speedup vs baseline: 6.3403x; 6.3403x over previous
import functools

import jax
import jax.numpy as jnp
from jax import lax
from jax.experimental import pallas as pl
from jax.experimental.pallas import tpu as pltpu
from jax.experimental.pallas import tpu_sc as plsc

NUM_EMBEDDINGS = 100000
EMBED_DIM = 128
PAD_INDEX = 0
BATCH = 4096
HIST_LEN = 200

_INFO = plsc.get_sparse_core_info()
_NC = _INFO.num_cores
_NS = _INFO.num_subcores
_NW = _NC * _NS
_B = BATCH * HIST_LEN
_PER_W = _B // _NW
_CHUNK = 128
_NCHUNK = _PER_W // _CHUNK


def _fixup_pad_rows(idx_v, buf, t):
    base = t * _CHUNK
    acc = jnp.zeros((16,), jnp.int32)
    for g in range(_CHUNK // 16):
        v = idx_v[pl.ds(base + 16 * g, 16)]
        acc = acc + jnp.minimum(v, 1)
    padcnt = plsc.all_reduce_population_count(acc < (_CHUNK // 16))

    @pl.when(padcnt[0] > 0)
    def _():
        zeros16 = jnp.zeros((16,), jnp.float32)
        for g in range(_CHUNK // 16):
            v = idx_v[pl.ds(base + 16 * g, 16)]
            m = v == PAD_INDEX
            rows = lax.iota(jnp.int32, 16) + 16 * g

            gcnt = plsc.all_reduce_population_count(m)

            @pl.when(gcnt[0] > 0)
            def _():
                def col_body(c, carry):
                    cols = jnp.zeros((16,), jnp.int32) + c
                    plsc.store_scatter(buf, [rows, cols], zeros16, mask=m)
                    return carry

                lax.fori_loop(0, EMBED_DIM, col_body, 0)


def _sc_kernel(idx_hbm, table_hbm, out_hbm, idx_v, buf, sem):
    wid = lax.axis_index("s") * _NC + lax.axis_index("c")
    wbase = wid * _PER_W
    pltpu.sync_copy(idx_hbm.at[pl.ds(wbase, _PER_W)], idx_v)

    def body(t, carry):
        cp = pltpu.async_copy(
            table_hbm.at[idx_v.at[pl.ds(t * _CHUNK, _CHUNK)]], buf, sem)
        cp.wait()
        _fixup_pad_rows(idx_v, buf, t)
        pltpu.sync_copy(buf, out_hbm.at[pl.ds(wbase + t * _CHUNK, _CHUNK)])
        return carry

    lax.fori_loop(0, _NCHUNK, body, 0)


@functools.partial(
    pl.kernel,
    out_type=jax.ShapeDtypeStruct((_B, EMBED_DIM), jnp.float32),
    mesh=plsc.VectorSubcoreMesh(core_axis_name="c", subcore_axis_name="s"),
    scratch_types=[
        pltpu.VMEM((_PER_W,), jnp.int32),
        pltpu.VMEM((_CHUNK, EMBED_DIM), jnp.float32),
        pltpu.SemaphoreType.DMA,
    ],
    compiler_params=pltpu.CompilerParams(needs_layout_passes=False),
)
def _embed_sc(idx_hbm, table_hbm, out_hbm, idx_v, buf, sem):
    _sc_kernel(idx_hbm, table_hbm, out_hbm, idx_v, buf, sem)


def kernel(indices, table):
    idx_flat = indices.reshape(_B).astype(jnp.int32)
    out = _embed_sc(idx_flat, table)
    return out.reshape(BATCH, HIST_LEN, EMBED_DIM)


# speedup vs baseline: 9.2639x; 1.4611x over previous
import functools

import jax
import jax.numpy as jnp
from jax import lax
from jax.experimental import pallas as pl
from jax.experimental.pallas import tpu as pltpu
from jax.experimental.pallas import tpu_sc as plsc

NUM_EMBEDDINGS = 100000
EMBED_DIM = 128
PAD_INDEX = 0
BATCH = 4096
HIST_LEN = 200

_INFO = plsc.get_sparse_core_info()
_NC = _INFO.num_cores
_NS = _INFO.num_subcores
_NW = _NC * _NS
_B = BATCH * HIST_LEN
_PER_W = _B // _NW
_CHUNK = 128
_NCHUNK = _PER_W // _CHUNK
_NSLOT = 4
_LOOKAHEAD = 2


def _fixup_pad_rows(idx_v, buf, t):
    base = t * _CHUNK
    acc = jnp.zeros((16,), jnp.int32)
    for g in range(_CHUNK // 16):
        v = idx_v[pl.ds(base + 16 * g, 16)]
        acc = acc + jnp.minimum(v, 1)
    padcnt = plsc.all_reduce_population_count(acc < (_CHUNK // 16))

    @pl.when(padcnt[0] > 0)
    def _():
        zeros16 = jnp.zeros((16,), jnp.float32)
        for g in range(_CHUNK // 16):
            v = idx_v[pl.ds(base + 16 * g, 16)]
            m = v == PAD_INDEX
            rows = lax.iota(jnp.int32, 16) + 16 * g
            gcnt = plsc.all_reduce_population_count(m)

            @pl.when(gcnt[0] > 0)
            def _():
                def col_body(c, carry):
                    cols = jnp.zeros((16,), jnp.int32) + c
                    plsc.store_scatter(buf, [rows, cols], zeros16, mask=m)
                    return carry

                lax.fori_loop(0, EMBED_DIM, col_body, 0)


def _sc_kernel(idx_hbm, table_hbm, out_hbm, idx_v, buf, gsem, osem):
    wid = lax.axis_index("s") * _NC + lax.axis_index("c")
    wbase = wid * _PER_W
    pltpu.sync_copy(idx_hbm.at[pl.ds(wbase, _PER_W)], idx_v)

    def gather_cp(t, slot):
        return pltpu.make_async_copy(
            table_hbm.at[idx_v.at[pl.ds(t * _CHUNK, _CHUNK)]],
            buf.at[slot], gsem.at[slot])

    def out_cp(t, slot):
        return pltpu.make_async_copy(
            buf.at[slot], out_hbm.at[pl.ds(wbase + t * _CHUNK, _CHUNK)],
            osem.at[slot])

    for t in range(_LOOKAHEAD):
        gather_cp(t, t).start()

    def grp_body(g, carry):
        for k in range(_NSLOT):
            t = _NSLOT * g + k
            nslot = (k + _LOOKAHEAD) % _NSLOT
            tn = t + _LOOKAHEAD

            @pl.when(tn < _NCHUNK)
            def _():
                @pl.when(tn >= _NSLOT)
                def _():
                    out_cp(tn - _NSLOT, nslot).wait()

                gather_cp(tn, nslot).start()

            gather_cp(t, k).wait()
            _fixup_pad_rows(idx_v, buf.at[k], t)
            out_cp(t, k).start()
        return carry

    lax.fori_loop(0, _NCHUNK // _NSLOT, grp_body, 0)

    for k in range(_NSLOT):
        out_cp(_NCHUNK - _NSLOT + k, k).wait()


@functools.partial(
    pl.kernel,
    out_type=jax.ShapeDtypeStruct((_B, EMBED_DIM), jnp.float32),
    mesh=plsc.VectorSubcoreMesh(core_axis_name="c", subcore_axis_name="s"),
    scratch_types=[
        pltpu.VMEM((_PER_W,), jnp.int32),
        pltpu.VMEM((_NSLOT, _CHUNK, EMBED_DIM), jnp.float32),
        pltpu.SemaphoreType.DMA((_NSLOT,)),
        pltpu.SemaphoreType.DMA((_NSLOT,)),
    ],
    compiler_params=pltpu.CompilerParams(needs_layout_passes=False),
)
def _embed_sc(idx_hbm, table_hbm, out_hbm, idx_v, buf, gsem, osem):
    _sc_kernel(idx_hbm, table_hbm, out_hbm, idx_v, buf, gsem, osem)


def kernel(indices, table):
    idx_flat = indices.reshape(_B).astype(jnp.int32)
    out = _embed_sc(idx_flat, table)
    return out.reshape(BATCH, HIST_LEN, EMBED_DIM)
